# Initial kernel scaffold; baseline (speedup 1.0000x reference)
import jax
import jax.numpy as jnp
from jax.experimental import pallas as pl


def kernel(encoder_outputs, durations, frames_positions, input_lengths):
    raise NotImplementedError("write your pallas kernel here")


# speedup vs baseline: 5.4578x; 5.4578x over previous
import functools

import jax
import jax.numpy as jnp
from jax import lax
from jax.experimental import pallas as pl
from jax.experimental.pallas import tpu as pltpu
from jax.experimental.pallas import tpu_sc as plsc

_NC = 2
_NS = 16
_LANES = 16
_CHUNK = 128


@functools.lru_cache(maxsize=None)
def _build(B, L, D, T):
    W = _NC * _NS
    WPB = W // B
    HALF = T // WPB
    NCHUNK = HALF // _CHUNK
    OUTD = D + 4
    ZROW = B * L
    SENT = jnp.int32(0x3FFFFFFF)

    mesh = plsc.VectorSubcoreMesh(
        core_axis_name="c", subcore_axis_name="s",
        num_cores=_NC, num_subcores=_NS)

    @functools.partial(
        pl.kernel,
        out_type=jax.ShapeDtypeStruct((B * T, OUTD), jnp.float32),
        mesh=mesh,
        compiler_params=pltpu.CompilerParams(needs_layout_passes=False),
        scratch_types=[
            pltpu.VMEM((L,), jnp.int32),
            pltpu.VMEM((L + _LANES,), jnp.int32),
            pltpu.VMEM((HALF,), jnp.int32),
            pltpu.VMEM((HALF,), jnp.int32),
            pltpu.VMEM((_CHUNK, 4), jnp.float32),
            pltpu.VMEM((_CHUNK, D), jnp.float32),
            pltpu.SemaphoreType.DMA,
        ],
    )
    def sc_expand(enc_hbm, dur_hbm, fr_hbm, out_hbm,
                  dur_v, cum_v, m_v, idx_v, fbuf, gbuf, sem):
        wid = lax.axis_index("s") * _NC + lax.axis_index("c")
        b = wid // WPB
        start_t = (wid % WPB) * HALF

        lane = lax.iota(jnp.int32, _LANES)

        pltpu.sync_copy(dur_hbm.at[b], dur_v)

        cum_v[pl.ds(L, _LANES)] = jnp.full((_LANES,), SENT, jnp.int32)

        def cs_body(j, run):
            x = dur_v[pl.ds(j * _LANES, _LANES)]
            s = plsc.cumsum(x) + run
            cum_v[pl.ds(j * _LANES, _LANES)] = s
            return jnp.max(s)

        mel_len = lax.fori_loop(0, L // _LANES, cs_body, jnp.int32(0))

        def z_body(i, _):
            m_v[pl.ds(i * _LANES, _LANES)] = jnp.zeros((_LANES,), jnp.int32)
            return 0

        lax.fori_loop(0, HALF // _LANES, z_body, 0)

        one = jnp.ones((_LANES,), jnp.int32)
        zero = jnp.zeros((_LANES,), jnp.int32)

        def sc_body(j, cnt):
            c16 = cum_v[pl.ds(j * _LANES, _LANES)]
            cnx = cum_v[pl.ds(j * _LANES + 1, _LANES)]
            cnt = cnt + jnp.sum(jnp.where(c16 < start_t, one, zero))
            pos = c16 - start_t
            keep = (c16 != cnx) & (pos >= 0) & (pos < HALF)
            vals = j * _LANES + lane + 1
            plsc.store_scatter(m_v, [pos], vals, mask=keep)
            return cnt

        seed = lax.fori_loop(0, L // _LANES, sc_body, jnp.int32(0))

        def mx_body(i, run):
            v = m_v[pl.ds(i * _LANES, _LANES)]
            s = jnp.maximum(plsc.cummax(v), run)
            t16 = start_t + i * _LANES + lane
            g = jnp.where(t16 < mel_len, b * L + s, jnp.int32(ZROW))
            idx_v[pl.ds(i * _LANES, _LANES)] = g
            return jnp.max(s)

        lax.fori_loop(0, HALF // _LANES, mx_body, seed)

        def g_body(c, _):
            row0 = c * _CHUNK
            orow = b * T + start_t + row0
            pltpu.async_copy(
                enc_hbm.at[idx_v.at[pl.ds(row0, _CHUNK)]], gbuf, sem).wait()
            pltpu.sync_copy(fr_hbm.at[b, pl.ds(start_t + row0, _CHUNK)], fbuf)
            pltpu.sync_copy(
                gbuf, out_hbm.at[pl.ds(orow, _CHUNK), pl.ds(0, D)])
            pltpu.sync_copy(
                fbuf, out_hbm.at[pl.ds(orow, _CHUNK), pl.ds(D, 4)])
            return 0

        lax.fori_loop(0, NCHUNK, g_body, 0)

    return sc_expand


def kernel(encoder_outputs, durations, frames_positions, input_lengths):
    B, L, D = encoder_outputs.shape
    T, DP = frames_positions.shape[1], frames_positions.shape[2]
    enc = jnp.pad(encoder_outputs.reshape(B * L, D), ((0, 8), (0, 0)))
    out = _build(B, L, D, T)(enc, durations, frames_positions)
    return out.reshape(B, T, D + DP)
